# baseline (speedup 1.0000x reference)
import functools

import jax
import jax.numpy as jnp
from jax import lax
from jax.experimental import pallas as pl
from jax.experimental.pallas import tpu as pltpu
from jax.experimental.pallas import tpu_sc as plsc

N_NODES = 10000
N_EDGES = 320000
D = 128
N_LAYERS = 4

NC = 2
NS = 16
NW = NC * NS
N_PAD = 10240
CHUNK = 128
EPT = 10240
NCHUNK = EPT // CHUNK
E_PAD = NW * EPT
ROWS_T = N_PAD // NS
DW = 16
BT = 10240
LN_EPS = 1e-6
NBUF = 4


DH = D // 2
HROWS_T = 2 * N_PAD // NS


def _msg_body(hb_hbm, slo_hbm, shi_hbm, dst_hbm, zero_hbm, out_hbm,
              slo_v, shi_v, dst_v, b0, b1, b2, b3,
              g0, g1, g2, g3, s0, s1, s2, s3, h_sh, agg_sh):
    c = lax.axis_index("c")
    s = lax.axis_index("s")
    wid = s * NC + c
    bufs = (b0, b1, b2, b3)
    gsem = (g0, g1, g2, g3)
    ssem = (s0, s1, s2, s3)
    pltpu.sync_copy(hb_hbm.at[pl.ds(s * HROWS_T, HROWS_T)],
                    h_sh.at[pl.ds(s * HROWS_T, HROWS_T)])
    pltpu.sync_copy(slo_hbm.at[wid], slo_v)
    pltpu.sync_copy(shi_hbm.at[wid], shi_v)
    pltpu.sync_copy(dst_hbm.at[wid], dst_v)

    for p, src_v in ((0, slo_v), (1, shi_v)):
        pltpu.sync_copy(zero_hbm.at[pl.ds(s * ROWS_T, ROWS_T)],
                        agg_sh.at[pl.ds(s * ROWS_T, ROWS_T)])
        plsc.subcore_barrier()

        for b in range(NBUF - 1):
            pltpu.async_copy(h_sh.at[src_v.at[b]], bufs[b], gsem[b])

        def body(g, carry):
            for b in range(NBUF):
                j = NBUF * g + b
                t = (b + NBUF - 1) % NBUF
                pltpu.make_async_copy(h_sh.at[src_v.at[j]],
                                      bufs[b], gsem[b]).wait()
                pltpu.async_copy(bufs[b], agg_sh.at[dst_v.at[j]],
                                 ssem[b], add=True)

                @pl.when(j + NBUF - 1 < NCHUNK)
                def _():
                    @pl.when(j >= 1)
                    def _():
                        pltpu.make_async_copy(
                            bufs[t], agg_sh.at[dst_v.at[0]], ssem[t]).wait()
                    pltpu.async_copy(h_sh.at[src_v.at[j + NBUF - 1]],
                                     bufs[t], gsem[t])
            return carry

        lax.fori_loop(0, NCHUNK // NBUF, body, 0)
        for b in range(NBUF):
            pltpu.make_async_copy(bufs[b], agg_sh.at[dst_v.at[0]],
                                  ssem[b]).wait()
        plsc.subcore_barrier()
        pltpu.sync_copy(agg_sh.at[pl.ds(s * ROWS_T, ROWS_T)],
                        out_hbm.at[p * NW + c * NS + s])


def _deg_body(dst_hbm, zero_hbm, ones_hbm, out_hbm, dst_v, ones_v, deg_sh):
    c = lax.axis_index("c")
    s = lax.axis_index("s")
    wid = s * NC + c
    pltpu.sync_copy(zero_hbm.at[pl.ds(s * ROWS_T, ROWS_T)],
                    deg_sh.at[pl.ds(s * ROWS_T, ROWS_T)])
    pltpu.sync_copy(dst_hbm.at[wid], dst_v)
    pltpu.sync_copy(ones_hbm, ones_v)
    plsc.subcore_barrier()

    def body(j, carry):
        pltpu.sync_copy(ones_v, deg_sh.at[dst_v.at[j]], add=True)
        return carry

    lax.fori_loop(0, NCHUNK, body, 0)
    plsc.subcore_barrier()
    pltpu.sync_copy(deg_sh.at[pl.ds(s * ROWS_T, ROWS_T)],
                    out_hbm.at[c * NS + s])


@functools.lru_cache(maxsize=None)
def _sc_kernels():
    mesh = plsc.VectorSubcoreMesh(core_axis_name="c", subcore_axis_name="s",
                                  num_cores=NC, num_subcores=NS)
    params = pltpu.CompilerParams(use_tc_tiling_on_sc=False)
    msg = pl.kernel(
        _msg_body,
        compiler_params=params,
        out_type=jax.ShapeDtypeStruct((2 * NW, ROWS_T, DH), jnp.bfloat16),
        mesh=mesh,
        scratch_types=[
            pltpu.VMEM((NCHUNK, CHUNK), jnp.int32),
            pltpu.VMEM((NCHUNK, CHUNK), jnp.int32),
            pltpu.VMEM((NCHUNK, CHUNK), jnp.int32),
            pltpu.VMEM((CHUNK, DH), jnp.bfloat16),
            pltpu.VMEM((CHUNK, DH), jnp.bfloat16),
            pltpu.VMEM((CHUNK, DH), jnp.bfloat16),
            pltpu.VMEM((CHUNK, DH), jnp.bfloat16),
            pltpu.SemaphoreType.DMA,
            pltpu.SemaphoreType.DMA,
            pltpu.SemaphoreType.DMA,
            pltpu.SemaphoreType.DMA,
            pltpu.SemaphoreType.DMA,
            pltpu.SemaphoreType.DMA,
            pltpu.SemaphoreType.DMA,
            pltpu.SemaphoreType.DMA,
            pltpu.VMEM_SHARED((2 * N_PAD, DH), jnp.bfloat16),
            pltpu.VMEM_SHARED((N_PAD, DH), jnp.bfloat16),
        ],
    )
    deg = pl.kernel(
        _deg_body,
        compiler_params=params,
        out_type=jax.ShapeDtypeStruct((NW, ROWS_T, DW), jnp.float32),
        mesh=mesh,
        scratch_types=[
            pltpu.VMEM((NCHUNK, CHUNK), jnp.int32),
            pltpu.VMEM((CHUNK, DW), jnp.float32),
            pltpu.VMEM_SHARED((N_PAD, DW), jnp.float32),
        ],
    )
    return msg, deg


def _lin_body(x_ref, w_ref, b_ref, ob_ref):
    y = (jnp.dot(x_ref[...], w_ref[...],
                 preferred_element_type=jnp.float32) + b_ref[...])
    ob_ref[...] = y.astype(jnp.bfloat16)


def _layer_math(h, a_ref, d_ref, w_ref, b_ref, sc_ref, bi_ref):
    deg = jnp.maximum(d_ref[0, :, :1] + d_ref[1, :, :1], 1.0)
    a = jnp.concatenate(
        [a_ref[0, 0].astype(jnp.float32) + a_ref[0, 1].astype(jnp.float32),
         a_ref[1, 0].astype(jnp.float32) + a_ref[1, 1].astype(jnp.float32)],
        axis=-1)
    z = h + a / deg
    y = (jnp.dot(z, w_ref[...], preferred_element_type=jnp.float32)
         + b_ref[...])
    mu = jnp.mean(y, axis=-1, keepdims=True)
    var = jnp.mean(jnp.square(y - mu), axis=-1, keepdims=True)
    yn = (y - mu) * lax.rsqrt(var + LN_EPS) * sc_ref[...] + bi_ref[...]
    return jnp.maximum(yn, 0.0) + h


def _layer_body(h_ref, a_ref, d_ref, w_ref, b_ref, sc_ref, bi_ref,
                ob_ref):
    o = _layer_math(h_ref[...].astype(jnp.float32), a_ref, d_ref,
                    w_ref, b_ref, sc_ref, bi_ref)
    ob_ref[...] = o.astype(jnp.bfloat16)


def _last_body(h_ref, a_ref, d_ref, w_ref, b_ref, sc_ref, bi_ref,
               wo_ref, bo_ref, o_ref, acc_ref):
    j = pl.program_id(0)
    hb = _layer_math(h_ref[...].astype(jnp.float32), a_ref, d_ref,
                     w_ref, b_ref, sc_ref, bi_ref)
    rows = j * BT + lax.broadcasted_iota(jnp.int32, (BT, 1), 0)
    hb = jnp.where(rows < N_NODES, hb, 0.0)
    part = jnp.sum(hb, axis=0, keepdims=True)

    @pl.when(j == 0)
    def _():
        acc_ref[...] = jnp.zeros_like(acc_ref)

    acc_ref[...] += part

    @pl.when(j == pl.num_programs(0) - 1)
    def _():
        o_ref[...] = (jnp.dot(acc_ref[...], wo_ref[...],
                              preferred_element_type=jnp.float32)
                      * (1.0 / N_NODES) + bo_ref[...])


_GRID = N_PAD // BT
_blk = pl.BlockSpec((BT, D), lambda j: (j, 0))
_wblk = pl.BlockSpec((D, D), lambda j: (0, 0))
_vblk = pl.BlockSpec((1, D), lambda j: (0, 0))
_ablk = pl.BlockSpec((2, NC, BT, DH), lambda j: (0, 0, j, 0))
_dblk = pl.BlockSpec((NC, BT, DW), lambda j: (0, j, 0))

_lin_call = pl.pallas_call(
    _lin_body, grid=(_GRID,),
    in_specs=[_blk, _wblk, _vblk],
    out_specs=_blk,
    out_shape=jax.ShapeDtypeStruct((N_PAD, D), jnp.bfloat16),
)

_layer_call = pl.pallas_call(
    _layer_body, grid=(_GRID,),
    in_specs=[_blk, _ablk, _dblk, _wblk, _vblk, _vblk, _vblk],
    out_specs=_blk,
    out_shape=jax.ShapeDtypeStruct((N_PAD, D), jnp.bfloat16),
)

_last_call = pl.pallas_call(
    _last_body, grid=(_GRID,),
    in_specs=[_blk, _ablk, _dblk, _wblk, _vblk, _vblk, _vblk, _wblk, _vblk],
    out_specs=pl.BlockSpec((1, D), lambda j: (0, 0)),
    out_shape=jax.ShapeDtypeStruct((1, D), jnp.float32),
    scratch_shapes=[pltpu.VMEM((1, D), jnp.float32)],
)


def kernel(x, edge_src, edge_dst, num_nodes, num_edges,
           w_in, b_in, w_conv, b_conv, ln_scale, ln_bias, w_out, b_out):
    del num_nodes, num_edges
    x = x.astype(jnp.float32)
    edge_src = edge_src.astype(jnp.int32)
    edge_dst = edge_dst.astype(jnp.int32)

    pad_e = E_PAD - N_EDGES
    x_pad = jnp.pad(x, ((0, N_PAD - N_NODES), (0, 0)))
    src_pad = jnp.concatenate(
        [edge_src, jnp.zeros((pad_e,), jnp.int32)]).reshape(NW, NCHUNK, CHUNK)
    dst_pad = jnp.concatenate(
        [edge_dst, jnp.full((pad_e,), N_NODES, jnp.int32)]
    ).reshape(NW, NCHUNK, CHUNK)
    src_lo = src_pad * 2
    src_hi = src_pad * 2 + 1
    zeros_b = jnp.zeros((N_PAD, DH), jnp.bfloat16)
    zeros_w = jnp.zeros((N_PAD, DW), jnp.float32)
    ones_w = jnp.ones((CHUNK, DW), jnp.float32)

    msg, deg_k = _sc_kernels()
    deg = deg_k(dst_pad, zeros_w, ones_w).reshape(NC, N_PAD, DW)

    hb = _lin_call(x_pad, w_in, b_in.reshape(1, D))
    for i in range(N_LAYERS):
        agg = msg(hb.reshape(2 * N_PAD, DH), src_lo, src_hi, dst_pad,
                  zeros_b).reshape(2, NC, N_PAD, DH)
        if i < N_LAYERS - 1:
            hb = _layer_call(hb, agg, deg, w_conv[i], b_conv[i].reshape(1, D),
                             ln_scale[i].reshape(1, D),
                             ln_bias[i].reshape(1, D))
        else:
            out = _last_call(hb, agg, deg, w_conv[i], b_conv[i].reshape(1, D),
                             ln_scale[i].reshape(1, D),
                             ln_bias[i].reshape(1, D),
                             w_out, b_out.reshape(1, D))
    return out.reshape(D)


# speedup vs baseline: 1.0212x; 1.0212x over previous
import functools

import jax
import jax.numpy as jnp
from jax import lax
from jax.experimental import pallas as pl
from jax.experimental.pallas import tpu as pltpu
from jax.experimental.pallas import tpu_sc as plsc

N_NODES = 10000
N_EDGES = 320000
D = 128
N_LAYERS = 4

NC = 2
NS = 16
NW = NC * NS
N_PAD = 10240
CHUNK = 128
EPT = 10240
NCHUNK = EPT // CHUNK
E_PAD = NW * EPT
ROWS_T = N_PAD // NS
DW = 16
BT = 5120
LN_EPS = 1e-6
NBUF = 4


DH = D // 2
HROWS_T = 2 * N_PAD // NS


def _msg_body(hb_hbm, slo_hbm, shi_hbm, dst_hbm, zero_hbm, out_hbm,
              slo_v, shi_v, dst_v, b0, b1, b2, b3,
              g0, g1, g2, g3, s0, s1, s2, s3, h_sh, agg_sh):
    c = lax.axis_index("c")
    s = lax.axis_index("s")
    wid = s * NC + c
    bufs = (b0, b1, b2, b3)
    gsem = (g0, g1, g2, g3)
    ssem = (s0, s1, s2, s3)
    pltpu.sync_copy(hb_hbm.at[pl.ds(s * HROWS_T, HROWS_T)],
                    h_sh.at[pl.ds(s * HROWS_T, HROWS_T)])
    pltpu.sync_copy(slo_hbm.at[wid], slo_v)
    pltpu.sync_copy(shi_hbm.at[wid], shi_v)
    pltpu.sync_copy(dst_hbm.at[wid], dst_v)

    for p, src_v in ((0, slo_v), (1, shi_v)):
        pltpu.sync_copy(zero_hbm.at[pl.ds(s * ROWS_T, ROWS_T)],
                        agg_sh.at[pl.ds(s * ROWS_T, ROWS_T)])
        plsc.subcore_barrier()

        for b in range(NBUF - 1):
            pltpu.async_copy(h_sh.at[src_v.at[b]], bufs[b], gsem[b])

        def body(g, carry):
            for b in range(NBUF):
                j = NBUF * g + b
                t = (b + NBUF - 1) % NBUF
                pltpu.make_async_copy(h_sh.at[src_v.at[j]],
                                      bufs[b], gsem[b]).wait()
                pltpu.async_copy(bufs[b], agg_sh.at[dst_v.at[j]],
                                 ssem[b], add=True)

                @pl.when(j + NBUF - 1 < NCHUNK)
                def _():
                    @pl.when(j >= 1)
                    def _():
                        pltpu.make_async_copy(
                            bufs[t], agg_sh.at[dst_v.at[0]], ssem[t]).wait()
                    pltpu.async_copy(h_sh.at[src_v.at[j + NBUF - 1]],
                                     bufs[t], gsem[t])
            return carry

        lax.fori_loop(0, NCHUNK // NBUF, body, 0)
        for b in range(NBUF):
            pltpu.make_async_copy(bufs[b], agg_sh.at[dst_v.at[0]],
                                  ssem[b]).wait()
        plsc.subcore_barrier()
        pltpu.sync_copy(agg_sh.at[pl.ds(s * ROWS_T, ROWS_T)],
                        out_hbm.at[p * NW + c * NS + s])


def _deg_body(dst_hbm, zero_hbm, ones_hbm, out_hbm, dst_v, ones_v, deg_sh):
    c = lax.axis_index("c")
    s = lax.axis_index("s")
    wid = s * NC + c
    pltpu.sync_copy(zero_hbm.at[pl.ds(s * ROWS_T, ROWS_T)],
                    deg_sh.at[pl.ds(s * ROWS_T, ROWS_T)])
    pltpu.sync_copy(dst_hbm.at[wid], dst_v)
    pltpu.sync_copy(ones_hbm, ones_v)
    plsc.subcore_barrier()

    def body(j, carry):
        pltpu.sync_copy(ones_v, deg_sh.at[dst_v.at[j]], add=True)
        return carry

    lax.fori_loop(0, NCHUNK, body, 0)
    plsc.subcore_barrier()
    pltpu.sync_copy(deg_sh.at[pl.ds(s * ROWS_T, ROWS_T)],
                    out_hbm.at[c * NS + s])


@functools.lru_cache(maxsize=None)
def _sc_kernels():
    mesh = plsc.VectorSubcoreMesh(core_axis_name="c", subcore_axis_name="s",
                                  num_cores=NC, num_subcores=NS)
    params = pltpu.CompilerParams(use_tc_tiling_on_sc=False)
    msg = pl.kernel(
        _msg_body,
        compiler_params=params,
        out_type=jax.ShapeDtypeStruct((2 * NW, ROWS_T, DH), jnp.bfloat16),
        mesh=mesh,
        scratch_types=[
            pltpu.VMEM((NCHUNK, CHUNK), jnp.int32),
            pltpu.VMEM((NCHUNK, CHUNK), jnp.int32),
            pltpu.VMEM((NCHUNK, CHUNK), jnp.int32),
            pltpu.VMEM((CHUNK, DH), jnp.bfloat16),
            pltpu.VMEM((CHUNK, DH), jnp.bfloat16),
            pltpu.VMEM((CHUNK, DH), jnp.bfloat16),
            pltpu.VMEM((CHUNK, DH), jnp.bfloat16),
            pltpu.SemaphoreType.DMA,
            pltpu.SemaphoreType.DMA,
            pltpu.SemaphoreType.DMA,
            pltpu.SemaphoreType.DMA,
            pltpu.SemaphoreType.DMA,
            pltpu.SemaphoreType.DMA,
            pltpu.SemaphoreType.DMA,
            pltpu.SemaphoreType.DMA,
            pltpu.VMEM_SHARED((2 * N_PAD, DH), jnp.bfloat16),
            pltpu.VMEM_SHARED((N_PAD, DH), jnp.bfloat16),
        ],
    )
    deg = pl.kernel(
        _deg_body,
        compiler_params=params,
        out_type=jax.ShapeDtypeStruct((NW, ROWS_T, DW), jnp.float32),
        mesh=mesh,
        scratch_types=[
            pltpu.VMEM((NCHUNK, CHUNK), jnp.int32),
            pltpu.VMEM((CHUNK, DW), jnp.float32),
            pltpu.VMEM_SHARED((N_PAD, DW), jnp.float32),
        ],
    )
    return msg, deg


def _lin_body(x_ref, w_ref, b_ref, ob_ref):
    y = (jnp.dot(x_ref[...], w_ref[...],
                 preferred_element_type=jnp.float32) + b_ref[...])
    ob_ref[...] = y.astype(jnp.bfloat16)


def _layer_math(h, a_ref, d_ref, w_ref, b_ref, sc_ref, bi_ref):
    deg = jnp.maximum(d_ref[0, :, :1] + d_ref[1, :, :1], 1.0)
    a = jnp.concatenate(
        [a_ref[0, 0].astype(jnp.float32) + a_ref[0, 1].astype(jnp.float32),
         a_ref[1, 0].astype(jnp.float32) + a_ref[1, 1].astype(jnp.float32)],
        axis=-1)
    z = h + a / deg
    y = (jnp.dot(z, w_ref[...], preferred_element_type=jnp.float32)
         + b_ref[...])
    mu = jnp.mean(y, axis=-1, keepdims=True)
    var = jnp.mean(jnp.square(y - mu), axis=-1, keepdims=True)
    yn = (y - mu) * lax.rsqrt(var + LN_EPS) * sc_ref[...] + bi_ref[...]
    return jnp.maximum(yn, 0.0) + h


def _layer_body(h_ref, a_ref, d_ref, w_ref, b_ref, sc_ref, bi_ref,
                ob_ref):
    o = _layer_math(h_ref[...].astype(jnp.float32), a_ref, d_ref,
                    w_ref, b_ref, sc_ref, bi_ref)
    ob_ref[...] = o.astype(jnp.bfloat16)


def _last_body(h_ref, a_ref, d_ref, w_ref, b_ref, sc_ref, bi_ref,
               wo_ref, bo_ref, o_ref, acc_ref):
    j = pl.program_id(0)
    hb = _layer_math(h_ref[...].astype(jnp.float32), a_ref, d_ref,
                     w_ref, b_ref, sc_ref, bi_ref)
    rows = j * BT + lax.broadcasted_iota(jnp.int32, (BT, 1), 0)
    hb = jnp.where(rows < N_NODES, hb, 0.0)
    part = jnp.sum(hb, axis=0, keepdims=True)

    @pl.when(j == 0)
    def _():
        acc_ref[...] = jnp.zeros_like(acc_ref)

    acc_ref[...] += part

    @pl.when(j == pl.num_programs(0) - 1)
    def _():
        o_ref[...] = (jnp.dot(acc_ref[...], wo_ref[...],
                              preferred_element_type=jnp.float32)
                      * (1.0 / N_NODES) + bo_ref[...])


_GRID = N_PAD // BT
_blk = pl.BlockSpec((BT, D), lambda j: (j, 0))
_wblk = pl.BlockSpec((D, D), lambda j: (0, 0))
_vblk = pl.BlockSpec((1, D), lambda j: (0, 0))
_ablk = pl.BlockSpec((2, NC, BT, DH), lambda j: (0, 0, j, 0))
_dblk = pl.BlockSpec((NC, BT, DW), lambda j: (0, j, 0))

_lin_call = pl.pallas_call(
    _lin_body, grid=(_GRID,),
    in_specs=[_blk, _wblk, _vblk],
    out_specs=_blk,
    out_shape=jax.ShapeDtypeStruct((N_PAD, D), jnp.bfloat16),
)

_layer_call = pl.pallas_call(
    _layer_body, grid=(_GRID,),
    in_specs=[_blk, _ablk, _dblk, _wblk, _vblk, _vblk, _vblk],
    out_specs=_blk,
    out_shape=jax.ShapeDtypeStruct((N_PAD, D), jnp.bfloat16),
)

_last_call = pl.pallas_call(
    _last_body, grid=(_GRID,),
    in_specs=[_blk, _ablk, _dblk, _wblk, _vblk, _vblk, _vblk, _wblk, _vblk],
    out_specs=pl.BlockSpec((1, D), lambda j: (0, 0)),
    out_shape=jax.ShapeDtypeStruct((1, D), jnp.float32),
    scratch_shapes=[pltpu.VMEM((1, D), jnp.float32)],
)


def kernel(x, edge_src, edge_dst, num_nodes, num_edges,
           w_in, b_in, w_conv, b_conv, ln_scale, ln_bias, w_out, b_out):
    del num_nodes, num_edges
    x = x.astype(jnp.float32)
    edge_src = edge_src.astype(jnp.int32)
    edge_dst = edge_dst.astype(jnp.int32)

    pad_e = E_PAD - N_EDGES
    x_pad = jnp.pad(x, ((0, N_PAD - N_NODES), (0, 0)))
    src_pad = jnp.concatenate(
        [edge_src, jnp.zeros((pad_e,), jnp.int32)]).reshape(NW, NCHUNK, CHUNK)
    dst_pad = jnp.concatenate(
        [edge_dst, jnp.full((pad_e,), N_NODES, jnp.int32)]
    ).reshape(NW, NCHUNK, CHUNK)
    src_lo = src_pad * 2
    src_hi = src_pad * 2 + 1
    zeros_b = jnp.zeros((N_PAD, DH), jnp.bfloat16)
    zeros_w = jnp.zeros((N_PAD, DW), jnp.float32)
    ones_w = jnp.ones((CHUNK, DW), jnp.float32)

    msg, deg_k = _sc_kernels()
    deg = deg_k(dst_pad, zeros_w, ones_w).reshape(NC, N_PAD, DW)

    hb = _lin_call(x_pad, w_in, b_in.reshape(1, D))
    for i in range(N_LAYERS):
        agg = msg(hb.reshape(2 * N_PAD, DH), src_lo, src_hi, dst_pad,
                  zeros_b).reshape(2, NC, N_PAD, DH)
        if i < N_LAYERS - 1:
            hb = _layer_call(hb, agg, deg, w_conv[i], b_conv[i].reshape(1, D),
                             ln_scale[i].reshape(1, D),
                             ln_bias[i].reshape(1, D))
        else:
            out = _last_call(hb, agg, deg, w_conv[i], b_conv[i].reshape(1, D),
                             ln_scale[i].reshape(1, D),
                             ln_bias[i].reshape(1, D),
                             w_out, b_out.reshape(1, D))
    return out.reshape(D)
